# baseline (speedup 1.0000x reference)
import functools

import jax
import jax.numpy as jnp
import numpy as np
from jax import lax
from jax.experimental import pallas as pl
from jax.experimental.pallas import tpu as pltpu
from jax.experimental.pallas import tpu_sc as plsc

N = 10000
E = 320000
D = 128
H = 128
C = 10
G = 64
EPS = 1e-5

NW = 32
WIN = 128
NWIN = 80
CH = 8
EPW = WIN * NWIN
EPAD = NW * EPW
NP = 10240
RPT = NP // 16
NBLK = 10
BLK = N // NBLK
ISQ = 1.0 / np.sqrt(1.0 + EPS)


@functools.cache
def _sc_kernels():
  mesh = plsc.VectorSubcoreMesh(core_axis_name="c", subcore_axis_name="s")

  def _agg_body(h_hbm, srcw_hbm, dstw_hbm, out_hbm, srcA, dstA, srcB, dstB,
                b0, b1, isemA, isemB, gs0, gs1, ss0, ss1, aggS, c, s, widx):
    bufs = (b0, b1)
    gsem = (gs0, gs1)
    ssem = (ss0, ss1)

    hi1 = pltpu.async_copy(srcw_hbm.at[widx, pl.ds(0, CH)], srcA, isemA)
    hi2 = pltpu.async_copy(dstw_hbm.at[widx, pl.ds(0, CH)], dstA, isemA)

    def _zb(i, carry):
        for k in range(H // 16):
            b0[i, pl.ds(k * 16, 16)] = jnp.zeros((16,), jnp.float32)
        return carry

    lax.fori_loop(0, WIN, _zb, 0)

    hz = [pltpu.async_copy(b0, aggS.at[pl.ds(s * RPT + t * WIN, WIN)], ss0)
          for t in range(RPT // WIN)]
    for h in hz:
        h.wait()
    hi1.wait()
    hi2.wait()

    plsc.subcore_barrier()

    def _gat(srcc, w, p):
        return (
            pltpu.async_copy(h_hbm.at[srcc.at[w, pl.ds(0, WIN // 2)]],
                             bufs[p].at[pl.ds(0, WIN // 2)], gsem[p]),
            pltpu.async_copy(h_hbm.at[srcc.at[w, pl.ds(WIN // 2, WIN // 2)]],
                             bufs[p].at[pl.ds(WIN // 2, WIN // 2)], gsem[p]),
        )

    def _proc(srcc, dstc):
        hg = [None, None]
        hs = [None, None]
        hg[0] = _gat(srcc, 0, 0)
        for w in range(CH):
            p = w % 2
            if w + 1 < CH:
                if hs[1 - p] is not None:
                    hs[1 - p].wait()
                hg[1 - p] = _gat(srcc, w + 1, 1 - p)
            hg[p][0].wait()
            hg[p][1].wait()
            hs[p] = pltpu.async_copy(bufs[p], aggS.at[dstc.at[w]], ssem[p],
                                     add=True)
        hs[0].wait()
        hs[1].wait()

    def _step(t, carry):
        c0 = t * 2 * CH
        hb1 = pltpu.async_copy(srcw_hbm.at[widx, pl.ds(c0 + CH, CH)], srcB,
                               isemB)
        hb2 = pltpu.async_copy(dstw_hbm.at[widx, pl.ds(c0 + CH, CH)], dstB,
                               isemB)
        _proc(srcA, dstA)
        hb1.wait()
        hb2.wait()
        nxt = jnp.minimum(c0 + 2 * CH, NWIN - CH)
        ha1 = pltpu.async_copy(srcw_hbm.at[widx, pl.ds(nxt, CH)], srcA, isemA)
        ha2 = pltpu.async_copy(dstw_hbm.at[widx, pl.ds(nxt, CH)], dstA, isemA)
        _proc(srcB, dstB)
        ha1.wait()
        ha2.wait()
        return carry

    lax.fori_loop(0, NWIN // (2 * CH), _step, 0)

    plsc.subcore_barrier()
    pltpu.sync_copy(aggS.at[pl.ds(s * RPT, RPT)],
                    out_hbm.at[c, pl.ds(s * RPT, RPT)])

  _SCRATCH = [
      pltpu.VMEM((CH, WIN), jnp.int32),
      pltpu.VMEM((CH, WIN), jnp.int32),
      pltpu.VMEM((CH, WIN), jnp.int32),
      pltpu.VMEM((CH, WIN), jnp.int32),
      pltpu.VMEM((WIN, H), jnp.float32),
      pltpu.VMEM((WIN, H), jnp.float32),
      pltpu.SemaphoreType.DMA,
      pltpu.SemaphoreType.DMA,
      pltpu.SemaphoreType.DMA,
      pltpu.SemaphoreType.DMA,
      pltpu.SemaphoreType.DMA,
      pltpu.SemaphoreType.DMA,
      pltpu.VMEM_SHARED((NP, H), jnp.float32),
  ]

  @functools.partial(
      pl.kernel,
      mesh=mesh,
      out_type=jax.ShapeDtypeStruct((2, NP, H), jnp.float32),
      scratch_types=list(_SCRATCH),
  )
  def _agg_sc(h_hbm, srcw_hbm, dstw_hbm, out_hbm, srcA, dstA, srcB, dstB,
              b0, b1, isemA, isemB, gs0, gs1, ss0, ss1, aggS):
    c = lax.axis_index("c")
    s = lax.axis_index("s")
    widx = c * 16 + s
    _agg_body(h_hbm, srcw_hbm, dstw_hbm, out_hbm, srcA, dstA, srcB, dstB,
              b0, b1, isemA, isemB, gs0, gs1, ss0, ss1, aggS, c, s, widx)

  @functools.partial(
      pl.kernel,
      mesh=mesh,
      out_type=(jax.ShapeDtypeStruct((2, NP, H), jnp.float32),
                jax.ShapeDtypeStruct((2, NP, H), jnp.float32)),
      scratch_types=list(_SCRATCH),
  )
  def _agg_deg_sc(h_hbm, srcw_hbm, dstw_hbm, agg_hbm, deg_hbm, srcA, dstA,
                  srcB, dstB, b0, b1, isemA, isemB, gs0, gs1, ss0, ss1, aggS):
    c = lax.axis_index("c")
    s = lax.axis_index("s")
    widx = c * 16 + s
    _agg_body(h_hbm, srcw_hbm, dstw_hbm, agg_hbm, srcA, dstA, srcB, dstB,
              b0, b1, isemA, isemB, gs0, gs1, ss0, ss1, aggS, c, s, widx)

    plsc.subcore_barrier()

    def _ob(i, carry):
        for k in range(H // 16):
            b1[i, pl.ds(k * 16, 16)] = jnp.ones((16,), jnp.float32)
        return carry

    lax.fori_loop(0, WIN, _ob, 0)

    def _zb(i, carry):
        for k in range(H // 16):
            b0[i, pl.ds(k * 16, 16)] = jnp.zeros((16,), jnp.float32)
        return carry

    lax.fori_loop(0, WIN, _zb, 0)

    hz = [pltpu.async_copy(b0, aggS.at[pl.ds(s * RPT + t * WIN, WIN)], ss0)
          for t in range(RPT // WIN)]
    for h in hz:
        h.wait()
    hi = pltpu.async_copy(dstw_hbm.at[widx, pl.ds(0, CH)], dstA, isemA)
    hi.wait()

    plsc.subcore_barrier()

    qsem = (gs0, gs1, ss0, ss1)

    def _dproc(dstc):
        hs = []
        for w in range(CH):
            if w >= 4:
                hs[w - 4].wait()
            hs.append(pltpu.async_copy(b1, aggS.at[dstc.at[w]], qsem[w % 4],
                                       add=True))
        for w in range(CH - 4, CH):
            hs[w].wait()

    def _dstep(t, carry):
        c0 = t * 2 * CH
        hb = pltpu.async_copy(dstw_hbm.at[widx, pl.ds(c0 + CH, CH)], dstB,
                              isemB)
        _dproc(dstA)
        hb.wait()
        nxt = jnp.minimum(c0 + 2 * CH, NWIN - CH)
        ha = pltpu.async_copy(dstw_hbm.at[widx, pl.ds(nxt, CH)], dstA, isemA)
        _dproc(dstB)
        ha.wait()
        return carry

    lax.fori_loop(0, NWIN // (2 * CH), _dstep, 0)

    plsc.subcore_barrier()
    pltpu.sync_copy(aggS.at[pl.ds(s * RPT, RPT)],
                    deg_hbm.at[c, pl.ds(s * RPT, RPT)])

  return _agg_sc, _agg_deg_sc


def _mm_body(x_ref, w_ref, b_ref, o_ref):
    v = (jnp.dot(x_ref[...], w_ref[...], preferred_element_type=jnp.float32)
         + b_ref[...])
    o_ref[0] = v
    o_ref[1] = v


def _mm_call(x, w, b):
    return pl.pallas_call(
        _mm_body,
        grid=(NBLK,),
        in_specs=[
            pl.BlockSpec((BLK, D), lambda j: (j, 0)),
            pl.BlockSpec((D, H), lambda j: (0, 0)),
            pl.BlockSpec((1, H), lambda j: (0, 0)),
        ],
        out_specs=pl.BlockSpec((2, BLK, H), lambda j: (0, j, 0)),
        out_shape=jax.ShapeDtypeStruct((2, N, H), jnp.float32),
    )(x, w, b)


def _mid0_body(a0, a1, d0, d1, g_ref, be_ref, w_ref, b_ref, o_ref, oi_ref):
    a = a0[0] + a1[0]
    d = d0[0] + d1[0]
    inv = 1.0 / jnp.maximum(d, 1.0)
    oi_ref[...] = inv
    f = jnp.maximum(a * inv * (g_ref[...] * ISQ) + be_ref[...], 0.0)
    v = jnp.dot(f, w_ref[...], preferred_element_type=jnp.float32) + b_ref[...]
    o_ref[0] = v
    o_ref[1] = v


def _mid0_call(parts, degp, g, be, w, b):
    return pl.pallas_call(
        _mid0_body,
        grid=(NBLK,),
        in_specs=[
            pl.BlockSpec((1, BLK, H), lambda j: (0, j, 0)),
            pl.BlockSpec((1, BLK, H), lambda j: (1, j, 0)),
            pl.BlockSpec((1, BLK, H), lambda j: (0, j, 0)),
            pl.BlockSpec((1, BLK, H), lambda j: (1, j, 0)),
            pl.BlockSpec((1, H), lambda j: (0, 0)),
            pl.BlockSpec((1, H), lambda j: (0, 0)),
            pl.BlockSpec((H, H), lambda j: (0, 0)),
            pl.BlockSpec((1, H), lambda j: (0, 0)),
        ],
        out_specs=[
            pl.BlockSpec((2, BLK, H), lambda j: (0, j, 0)),
            pl.BlockSpec((BLK, H), lambda j: (j, 0)),
        ],
        out_shape=[
            jax.ShapeDtypeStruct((2, N, H), jnp.float32),
            jax.ShapeDtypeStruct((N, H), jnp.float32),
        ],
    )(parts, parts, degp, degp, g.reshape(1, H), be.reshape(1, H), w,
      b.reshape(1, H))


def _mid_body(a0, a1, iv, g_ref, be_ref, w_ref, b_ref, o_ref):
    a = a0[0] + a1[0]
    f = jnp.maximum(a * iv[...] * (g_ref[...] * ISQ) + be_ref[...], 0.0)
    v = jnp.dot(f, w_ref[...], preferred_element_type=jnp.float32) + b_ref[...]
    o_ref[0] = v
    o_ref[1] = v


def _mid_call(parts, invb, g, be, w, b):
    return pl.pallas_call(
        _mid_body,
        grid=(NBLK,),
        in_specs=[
            pl.BlockSpec((1, BLK, H), lambda j: (0, j, 0)),
            pl.BlockSpec((1, BLK, H), lambda j: (1, j, 0)),
            pl.BlockSpec((BLK, H), lambda j: (j, 0)),
            pl.BlockSpec((1, H), lambda j: (0, 0)),
            pl.BlockSpec((1, H), lambda j: (0, 0)),
            pl.BlockSpec((H, H), lambda j: (0, 0)),
            pl.BlockSpec((1, H), lambda j: (0, 0)),
        ],
        out_specs=pl.BlockSpec((2, BLK, H), lambda j: (0, j, 0)),
        out_shape=jax.ShapeDtypeStruct((2, N, H), jnp.float32),
    )(parts, parts, invb, g.reshape(1, H), be.reshape(1, H), w,
      b.reshape(1, H))


def _head_body(a0, a1, iv, g_ref, be_ref, bt, wh1, bh1, wh2, bh2, o_ref,
               ps, cs):
    j = pl.program_id(0)

    @pl.when(j == 0)
    def _():
        ps[...] = jnp.zeros_like(ps)
        cs[...] = jnp.zeros_like(cs)

    a = a0[0] + a1[0]
    f = jnp.maximum(a * iv[...] * (g_ref[...] * ISQ) + be_ref[...], 0.0)
    bids = bt[0]
    mask = jnp.equal(
        lax.broadcasted_iota(jnp.int32, (G, BLK), 0),
        jnp.broadcast_to(bids, (G, BLK)),
    ).astype(jnp.float32)
    ps[...] += jnp.dot(mask, f, preferred_element_type=jnp.float32)
    cs[...] += jnp.dot(mask, jnp.ones((BLK, H), jnp.float32),
                       preferred_element_type=jnp.float32)

    @pl.when(j == NBLK - 1)
    def _():
        pooled = ps[...] / jnp.maximum(cs[...], 1.0)
        hdn = jnp.maximum(
            jnp.dot(pooled, wh1[...], preferred_element_type=jnp.float32)
            + bh1[...], 0.0)
        o_ref[...] = (
            jnp.dot(hdn, wh2[...], preferred_element_type=jnp.float32)
            + bh2[...]
        )


def _head_call(parts, invb, g, be, batchr, wh1, bh1, wh2p, bh2p):
    return pl.pallas_call(
        _head_body,
        grid=(NBLK,),
        in_specs=[
            pl.BlockSpec((1, BLK, H), lambda j: (0, j, 0)),
            pl.BlockSpec((1, BLK, H), lambda j: (1, j, 0)),
            pl.BlockSpec((BLK, H), lambda j: (j, 0)),
            pl.BlockSpec((1, H), lambda j: (0, 0)),
            pl.BlockSpec((1, H), lambda j: (0, 0)),
            pl.BlockSpec((1, 1, BLK), lambda j: (j, 0, 0)),
            pl.BlockSpec((H, H), lambda j: (0, 0)),
            pl.BlockSpec((1, H), lambda j: (0, 0)),
            pl.BlockSpec((H, H), lambda j: (0, 0)),
            pl.BlockSpec((1, H), lambda j: (0, 0)),
        ],
        out_specs=pl.BlockSpec((G, H), lambda j: (0, 0)),
        out_shape=jax.ShapeDtypeStruct((G, H), jnp.float32),
        scratch_shapes=[
            pltpu.VMEM((G, H), jnp.float32),
            pltpu.VMEM((G, H), jnp.float32),
        ],
    )(parts, parts, invb, g.reshape(1, H), be.reshape(1, H), batchr,
      wh1, bh1.reshape(1, H), wh2p, bh2p.reshape(1, H))


def kernel(x, edge_index, batch, W0, b0, g0, be0, W1, b1, g1, be1, W2, b2,
           g2, be2, W3, b3, g3, be3, Wh1, bh1, Wh2, bh2):
    src = edge_index[0]
    dst = edge_index[1]
    npad = EPAD - E
    pidx = jnp.arange(npad, dtype=jnp.int32)
    srcp = jnp.concatenate([src, pidx % N]).reshape(NW, NWIN, WIN)
    srcp = srcp + (jnp.arange(NW, dtype=jnp.int32) >= 16).astype(
        jnp.int32)[:, None, None] * N
    dstp = jnp.concatenate([dst, N + pidx % (NP - N)]).reshape(NW, NWIN, WIN)

    agg_sc, agg_deg_sc = _sc_kernels()
    h = _mm_call(x, W0, b0.reshape(1, H)).reshape(2 * N, H)

    layer = [(g0, be0), (g1, be1), (g2, be2), (g3, be3)]
    nxt = [(W1, b1), (W2, b2), (W3, b3)]
    for i in range(3):
        g, be = layer[i]
        w, b = nxt[i]
        if i == 0:
            parts, degp = agg_deg_sc(h, srcp, dstp)
            h, invb = _mid0_call(parts, degp, g, be, w, b)
        else:
            parts = agg_sc(h, srcp, dstp)
            h = _mid_call(parts, invb, g, be, w, b)
        h = h.reshape(2 * N, H)

    parts = agg_sc(h, srcp, dstp)
    batchr = batch.reshape(NBLK, 1, BLK)
    wh2p = jnp.pad(Wh2, ((0, 0), (0, H - C)))
    bh2p = jnp.pad(bh2, (0, H - C))
    out = _head_call(parts, invb, layer[3][0], layer[3][1], batchr, Wh1,
                     bh1, wh2p, bh2p)
    return out[:, :C]


# speedup vs baseline: 1.0115x; 1.0115x over previous
import functools

import jax
import jax.numpy as jnp
import numpy as np
from jax import lax
from jax.experimental import pallas as pl
from jax.experimental.pallas import tpu as pltpu
from jax.experimental.pallas import tpu_sc as plsc

N = 10000
E = 320000
D = 128
H = 128
C = 10
G = 64
EPS = 1e-5

NW = 32
WIN = 128
NWIN = 80
CH = 8
EPW = WIN * NWIN
EPAD = NW * EPW
NP = 10240
RPT = NP // 16
NBLK = 10
BLK = N // NBLK
ISQ = 1.0 / np.sqrt(1.0 + EPS)


@functools.cache
def _sc_kernels():
  mesh = plsc.VectorSubcoreMesh(core_axis_name="c", subcore_axis_name="s")

  def _agg_body(h_hbm, srcw_hbm, dstw_hbm, out_hbm, srcA, dstA, srcB, dstB,
                b0, b1, isemA, isemB, gs0, gs1, ss0, ss1, aggS, c, s, widx):
    bufs = (b0, b1)
    gsem = (gs0, gs1)
    ssem = (ss0, ss1)

    hi1 = pltpu.async_copy(srcw_hbm.at[widx, pl.ds(0, CH)], srcA, isemA)
    hi2 = pltpu.async_copy(dstw_hbm.at[widx, pl.ds(0, CH)], dstA, isemA)

    def _zb(i, carry):
        for k in range(H // 16):
            b0[i, pl.ds(k * 16, 16)] = jnp.zeros((16,), jnp.float32)
        return carry

    lax.fori_loop(0, WIN, _zb, 0)

    hz = [pltpu.async_copy(b0, aggS.at[pl.ds(s * RPT + t * WIN, WIN)], ss0)
          for t in range(RPT // WIN)]
    for h in hz:
        h.wait()
    hi1.wait()
    hi2.wait()

    plsc.subcore_barrier()

    def _gat(srcc, w, p):
        return (
            pltpu.async_copy(h_hbm.at[srcc.at[w, pl.ds(0, WIN // 2)]],
                             bufs[p].at[pl.ds(0, WIN // 2)], gsem[p]),
            pltpu.async_copy(h_hbm.at[srcc.at[w, pl.ds(WIN // 2, WIN // 2)]],
                             bufs[p].at[pl.ds(WIN // 2, WIN // 2)], gsem[p]),
        )

    def _proc(srcc, dstc):
        hg = [None, None]
        hs = [None, None]
        hg[0] = _gat(srcc, 0, 0)
        for w in range(CH):
            p = w % 2
            if w + 1 < CH:
                if hs[1 - p] is not None:
                    hs[1 - p].wait()
                hg[1 - p] = _gat(srcc, w + 1, 1 - p)
            hg[p][0].wait()
            hg[p][1].wait()
            hs[p] = pltpu.async_copy(bufs[p], aggS.at[dstc.at[w]], ssem[p],
                                     add=True)
        hs[0].wait()
        hs[1].wait()

    def _step(t, carry):
        c0 = t * 2 * CH
        hb1 = pltpu.async_copy(srcw_hbm.at[widx, pl.ds(c0 + CH, CH)], srcB,
                               isemB)
        hb2 = pltpu.async_copy(dstw_hbm.at[widx, pl.ds(c0 + CH, CH)], dstB,
                               isemB)
        _proc(srcA, dstA)
        hb1.wait()
        hb2.wait()
        nxt = jnp.minimum(c0 + 2 * CH, NWIN - CH)
        ha1 = pltpu.async_copy(srcw_hbm.at[widx, pl.ds(nxt, CH)], srcA, isemA)
        ha2 = pltpu.async_copy(dstw_hbm.at[widx, pl.ds(nxt, CH)], dstA, isemA)
        _proc(srcB, dstB)
        ha1.wait()
        ha2.wait()
        return carry

    lax.fori_loop(0, NWIN // (2 * CH), _step, 0)

    plsc.subcore_barrier()
    pltpu.sync_copy(aggS.at[pl.ds(s * RPT, RPT)],
                    out_hbm.at[c, pl.ds(s * RPT, RPT)])

  _SCRATCH = [
      pltpu.VMEM((CH, WIN), jnp.int32),
      pltpu.VMEM((CH, WIN), jnp.int32),
      pltpu.VMEM((CH, WIN), jnp.int32),
      pltpu.VMEM((CH, WIN), jnp.int32),
      pltpu.VMEM((WIN, H), jnp.float32),
      pltpu.VMEM((WIN, H), jnp.float32),
      pltpu.SemaphoreType.DMA,
      pltpu.SemaphoreType.DMA,
      pltpu.SemaphoreType.DMA,
      pltpu.SemaphoreType.DMA,
      pltpu.SemaphoreType.DMA,
      pltpu.SemaphoreType.DMA,
      pltpu.VMEM_SHARED((NP, H), jnp.float32),
  ]

  @functools.partial(
      pl.kernel,
      mesh=mesh,
      out_type=jax.ShapeDtypeStruct((2, NP, H), jnp.float32),
      scratch_types=list(_SCRATCH),
  )
  def _agg_sc(h_hbm, srcw_hbm, dstw_hbm, out_hbm, srcA, dstA, srcB, dstB,
              b0, b1, isemA, isemB, gs0, gs1, ss0, ss1, aggS):
    c = lax.axis_index("c")
    s = lax.axis_index("s")
    widx = c * 16 + s
    _agg_body(h_hbm, srcw_hbm, dstw_hbm, out_hbm, srcA, dstA, srcB, dstB,
              b0, b1, isemA, isemB, gs0, gs1, ss0, ss1, aggS, c, s, widx)

  @functools.partial(
      pl.kernel,
      mesh=mesh,
      out_type=(jax.ShapeDtypeStruct((2, NP, H), jnp.float32),
                jax.ShapeDtypeStruct((2, NP, H), jnp.float32)),
      scratch_types=list(_SCRATCH),
  )
  def _agg_deg_sc(h_hbm, srcw_hbm, dstw_hbm, agg_hbm, deg_hbm, srcA, dstA,
                  srcB, dstB, b0, b1, isemA, isemB, gs0, gs1, ss0, ss1, aggS):
    c = lax.axis_index("c")
    s = lax.axis_index("s")
    widx = c * 16 + s
    _agg_body(h_hbm, srcw_hbm, dstw_hbm, agg_hbm, srcA, dstA, srcB, dstB,
              b0, b1, isemA, isemB, gs0, gs1, ss0, ss1, aggS, c, s, widx)

    plsc.subcore_barrier()

    def _ob(i, carry):
        for k in range(H // 16):
            b1[i, pl.ds(k * 16, 16)] = jnp.ones((16,), jnp.float32)
        return carry

    lax.fori_loop(0, WIN, _ob, 0)

    def _zb(i, carry):
        for k in range(H // 16):
            b0[i, pl.ds(k * 16, 16)] = jnp.zeros((16,), jnp.float32)
        return carry

    lax.fori_loop(0, WIN, _zb, 0)

    hz = [pltpu.async_copy(b0, aggS.at[pl.ds(s * RPT + t * WIN, WIN)], ss0)
          for t in range(RPT // WIN)]
    for h in hz:
        h.wait()
    hi = pltpu.async_copy(dstw_hbm.at[widx, pl.ds(0, CH)], dstA, isemA)
    hi.wait()

    plsc.subcore_barrier()

    qsem = (gs0, gs1, ss0, ss1)

    def _dproc(dstc):
        hs = []
        for w in range(CH):
            if w >= 4:
                hs[w - 4].wait()
            hs.append(pltpu.async_copy(b1, aggS.at[dstc.at[w]], qsem[w % 4],
                                       add=True))
        for w in range(CH - 4, CH):
            hs[w].wait()

    def _dstep(t, carry):
        c0 = t * 2 * CH
        hb = pltpu.async_copy(dstw_hbm.at[widx, pl.ds(c0 + CH, CH)], dstB,
                              isemB)
        _dproc(dstA)
        hb.wait()
        nxt = jnp.minimum(c0 + 2 * CH, NWIN - CH)
        ha = pltpu.async_copy(dstw_hbm.at[widx, pl.ds(nxt, CH)], dstA, isemA)
        _dproc(dstB)
        ha.wait()
        return carry

    lax.fori_loop(0, NWIN // (2 * CH), _dstep, 0)

    plsc.subcore_barrier()
    pltpu.sync_copy(aggS.at[pl.ds(s * RPT, RPT)],
                    deg_hbm.at[c, pl.ds(s * RPT, RPT)])

  return _agg_sc, _agg_deg_sc


def _mm_body(x_ref, w_ref, b_ref, o_ref):
    o_ref[...] = (
        jnp.dot(x_ref[...], w_ref[...], preferred_element_type=jnp.float32)
        + b_ref[...]
    )


def _mm_call(x, w, b):
    return pl.pallas_call(
        _mm_body,
        grid=(NBLK,),
        in_specs=[
            pl.BlockSpec((BLK, D), lambda j: (j, 0)),
            pl.BlockSpec((D, H), lambda j: (0, 0)),
            pl.BlockSpec((1, H), lambda j: (0, 0)),
        ],
        out_specs=pl.BlockSpec((BLK, H), lambda j: (j, 0)),
        out_shape=jax.ShapeDtypeStruct((N, H), jnp.float32),
    )(x, w, b)


def _mid0_body(a0, a1, d0, d1, g_ref, be_ref, w_ref, b_ref, o_ref, oi_ref):
    a = a0[0] + a1[0]
    d = d0[0] + d1[0]
    inv = 1.0 / jnp.maximum(d, 1.0)
    oi_ref[...] = inv
    f = jnp.maximum(a * inv * (g_ref[...] * ISQ) + be_ref[...], 0.0)
    o_ref[...] = (
        jnp.dot(f, w_ref[...], preferred_element_type=jnp.float32) + b_ref[...]
    )


def _mid0_call(parts, degp, g, be, w, b):
    return pl.pallas_call(
        _mid0_body,
        grid=(NBLK,),
        in_specs=[
            pl.BlockSpec((1, BLK, H), lambda j: (0, j, 0)),
            pl.BlockSpec((1, BLK, H), lambda j: (1, j, 0)),
            pl.BlockSpec((1, BLK, H), lambda j: (0, j, 0)),
            pl.BlockSpec((1, BLK, H), lambda j: (1, j, 0)),
            pl.BlockSpec((1, H), lambda j: (0, 0)),
            pl.BlockSpec((1, H), lambda j: (0, 0)),
            pl.BlockSpec((H, H), lambda j: (0, 0)),
            pl.BlockSpec((1, H), lambda j: (0, 0)),
        ],
        out_specs=[
            pl.BlockSpec((BLK, H), lambda j: (j, 0)),
            pl.BlockSpec((BLK, H), lambda j: (j, 0)),
        ],
        out_shape=[
            jax.ShapeDtypeStruct((N, H), jnp.float32),
            jax.ShapeDtypeStruct((N, H), jnp.float32),
        ],
    )(parts, parts, degp, degp, g.reshape(1, H), be.reshape(1, H), w,
      b.reshape(1, H))


def _mid_body(a0, a1, iv, g_ref, be_ref, w_ref, b_ref, o_ref):
    a = a0[0] + a1[0]
    f = jnp.maximum(a * iv[...] * (g_ref[...] * ISQ) + be_ref[...], 0.0)
    o_ref[...] = (
        jnp.dot(f, w_ref[...], preferred_element_type=jnp.float32) + b_ref[...]
    )


def _mid_call(parts, invb, g, be, w, b):
    return pl.pallas_call(
        _mid_body,
        grid=(NBLK,),
        in_specs=[
            pl.BlockSpec((1, BLK, H), lambda j: (0, j, 0)),
            pl.BlockSpec((1, BLK, H), lambda j: (1, j, 0)),
            pl.BlockSpec((BLK, H), lambda j: (j, 0)),
            pl.BlockSpec((1, H), lambda j: (0, 0)),
            pl.BlockSpec((1, H), lambda j: (0, 0)),
            pl.BlockSpec((H, H), lambda j: (0, 0)),
            pl.BlockSpec((1, H), lambda j: (0, 0)),
        ],
        out_specs=pl.BlockSpec((BLK, H), lambda j: (j, 0)),
        out_shape=jax.ShapeDtypeStruct((N, H), jnp.float32),
    )(parts, parts, invb, g.reshape(1, H), be.reshape(1, H), w,
      b.reshape(1, H))


def _head_body(a0, a1, iv, g_ref, be_ref, bt, wh1, bh1, wh2, bh2, o_ref,
               ps, cs):
    j = pl.program_id(0)

    @pl.when(j == 0)
    def _():
        ps[...] = jnp.zeros_like(ps)
        cs[...] = jnp.zeros_like(cs)

    a = a0[0] + a1[0]
    f = jnp.maximum(a * iv[...] * (g_ref[...] * ISQ) + be_ref[...], 0.0)
    bids = bt[0]
    mask = jnp.equal(
        lax.broadcasted_iota(jnp.int32, (G, BLK), 0),
        jnp.broadcast_to(bids, (G, BLK)),
    ).astype(jnp.float32)
    ps[...] += jnp.dot(mask, f, preferred_element_type=jnp.float32)
    cs[...] += jnp.dot(mask, jnp.ones((BLK, H), jnp.float32),
                       preferred_element_type=jnp.float32)

    @pl.when(j == NBLK - 1)
    def _():
        pooled = ps[...] / jnp.maximum(cs[...], 1.0)
        hdn = jnp.maximum(
            jnp.dot(pooled, wh1[...], preferred_element_type=jnp.float32)
            + bh1[...], 0.0)
        o_ref[...] = (
            jnp.dot(hdn, wh2[...], preferred_element_type=jnp.float32)
            + bh2[...]
        )


def _head_call(parts, invb, g, be, batchr, wh1, bh1, wh2p, bh2p):
    return pl.pallas_call(
        _head_body,
        grid=(NBLK,),
        in_specs=[
            pl.BlockSpec((1, BLK, H), lambda j: (0, j, 0)),
            pl.BlockSpec((1, BLK, H), lambda j: (1, j, 0)),
            pl.BlockSpec((BLK, H), lambda j: (j, 0)),
            pl.BlockSpec((1, H), lambda j: (0, 0)),
            pl.BlockSpec((1, H), lambda j: (0, 0)),
            pl.BlockSpec((1, 1, BLK), lambda j: (j, 0, 0)),
            pl.BlockSpec((H, H), lambda j: (0, 0)),
            pl.BlockSpec((1, H), lambda j: (0, 0)),
            pl.BlockSpec((H, H), lambda j: (0, 0)),
            pl.BlockSpec((1, H), lambda j: (0, 0)),
        ],
        out_specs=pl.BlockSpec((G, H), lambda j: (0, 0)),
        out_shape=jax.ShapeDtypeStruct((G, H), jnp.float32),
        scratch_shapes=[
            pltpu.VMEM((G, H), jnp.float32),
            pltpu.VMEM((G, H), jnp.float32),
        ],
    )(parts, parts, invb, g.reshape(1, H), be.reshape(1, H), batchr,
      wh1, bh1.reshape(1, H), wh2p, bh2p.reshape(1, H))


def kernel(x, edge_index, batch, W0, b0, g0, be0, W1, b1, g1, be1, W2, b2,
           g2, be2, W3, b3, g3, be3, Wh1, bh1, Wh2, bh2):
    src = edge_index[0]
    dst = edge_index[1]
    npad = EPAD - E
    pidx = jnp.arange(npad, dtype=jnp.int32)
    srcp = jnp.concatenate([src, pidx % N]).reshape(NW, NWIN, WIN)
    dstp = jnp.concatenate([dst, N + pidx % (NP - N)]).reshape(NW, NWIN, WIN)

    agg_sc, agg_deg_sc = _sc_kernels()
    h = _mm_call(x, W0, b0.reshape(1, H))

    layer = [(g0, be0), (g1, be1), (g2, be2), (g3, be3)]
    nxt = [(W1, b1), (W2, b2), (W3, b3)]
    for i in range(3):
        g, be = layer[i]
        w, b = nxt[i]
        if i == 0:
            parts, degp = agg_deg_sc(h, srcp, dstp)
            h, invb = _mid0_call(parts, degp, g, be, w, b)
        else:
            parts = agg_sc(h, srcp, dstp)
            h = _mid_call(parts, invb, g, be, w, b)

    parts = agg_sc(h, srcp, dstp)
    batchr = batch.reshape(NBLK, 1, BLK)
    wh2p = jnp.pad(Wh2, ((0, 0), (0, H - C)))
    bh2p = jnp.pad(bh2, (0, H - C))
    out = _head_call(parts, invb, layer[3][0], layer[3][1], batchr, Wh1,
                     bh1, wh2p, bh2p)
    return out[:, :C]


# speedup vs baseline: 1.0534x; 1.0414x over previous
import functools

import jax
import jax.numpy as jnp
import numpy as np
from jax import lax
from jax.experimental import pallas as pl
from jax.experimental.pallas import tpu as pltpu
from jax.experimental.pallas import tpu_sc as plsc

N = 10000
E = 320000
D = 128
H = 128
C = 10
G = 64
EPS = 1e-5

NW = 32
WIN = 128
NWIN = 80
CH = 8
EPW = WIN * NWIN
EPAD = NW * EPW
NP = 10240
RPT = NP // 16
NBLK = 10
BLK = N // NBLK
ISQ = 1.0 / np.sqrt(1.0 + EPS)


@functools.cache
def _sc_kernels():
  mesh = plsc.VectorSubcoreMesh(core_axis_name="c", subcore_axis_name="s")

  def _agg_body(h_hbm, srcw_hbm, dstw_hbm, out_hbm, srcA, dstA, srcB, dstB,
                b0, b1, isemA, isemB, gs0, gs1, ss0, ss1, aggS, c, s, widx):
    bufs = (b0, b1)
    gsem = (gs0, gs1)
    ssem = (ss0, ss1)

    hi1 = pltpu.async_copy(srcw_hbm.at[widx, pl.ds(0, CH)], srcA, isemA)
    hi2 = pltpu.async_copy(dstw_hbm.at[widx, pl.ds(0, CH)], dstA, isemA)

    def _zb(i, carry):
        for k in range(H // 16):
            b0[i, pl.ds(k * 16, 16)] = jnp.zeros((16,), jnp.float32)
        return carry

    lax.fori_loop(0, WIN, _zb, 0)

    hz = [pltpu.async_copy(b0, aggS.at[pl.ds(s * RPT + t * WIN, WIN)], ss0)
          for t in range(RPT // WIN)]
    for h in hz:
        h.wait()
    hi1.wait()
    hi2.wait()

    plsc.subcore_barrier()

    def _gat(srcc, w, p):
        return (
            pltpu.async_copy(h_hbm.at[srcc.at[w, pl.ds(0, WIN // 2)]],
                             bufs[p].at[pl.ds(0, WIN // 2)], gsem[p]),
            pltpu.async_copy(h_hbm.at[srcc.at[w, pl.ds(WIN // 2, WIN // 2)]],
                             bufs[p].at[pl.ds(WIN // 2, WIN // 2)], gsem[p]),
        )

    def _step(t, carry):
        c0 = t * 2 * CH
        hb1 = pltpu.async_copy(srcw_hbm.at[widx, pl.ds(c0 + CH, CH)], srcB,
                               isemB)
        hb2 = pltpu.async_copy(dstw_hbm.at[widx, pl.ds(c0 + CH, CH)], dstB,
                               isemB)
        nxt = jnp.minimum(c0 + 2 * CH, NWIN - CH)
        ha = [None, None]
        hg = [None, None]
        hs = [None, None]
        hg[0] = _gat(srcA, 0, 0)
        for w in range(2 * CH):
            p = w % 2
            if w + 1 < 2 * CH:
                if w + 1 == CH:
                    hb1.wait()
                    hb2.wait()
                if hs[1 - p] is not None:
                    hs[1 - p].wait()
                nsrc = srcA if w + 1 < CH else srcB
                hg[1 - p] = _gat(nsrc, (w + 1) % CH, 1 - p)
            if w == CH:
                ha[0] = pltpu.async_copy(srcw_hbm.at[widx, pl.ds(nxt, CH)],
                                         srcA, isemA)
                ha[1] = pltpu.async_copy(dstw_hbm.at[widx, pl.ds(nxt, CH)],
                                         dstA, isemA)
            hg[p][0].wait()
            hg[p][1].wait()
            dstc = dstA if w < CH else dstB
            hs[p] = pltpu.async_copy(bufs[p], aggS.at[dstc.at[w % CH]],
                                     ssem[p], add=True)
        hs[0].wait()
        hs[1].wait()
        ha[0].wait()
        ha[1].wait()
        return carry

    lax.fori_loop(0, NWIN // (2 * CH), _step, 0)

    plsc.subcore_barrier()
    pltpu.sync_copy(aggS.at[pl.ds(s * RPT, RPT)],
                    out_hbm.at[c, pl.ds(s * RPT, RPT)])

  _SCRATCH = [
      pltpu.VMEM((CH, WIN), jnp.int32),
      pltpu.VMEM((CH, WIN), jnp.int32),
      pltpu.VMEM((CH, WIN), jnp.int32),
      pltpu.VMEM((CH, WIN), jnp.int32),
      pltpu.VMEM((WIN, H), jnp.float32),
      pltpu.VMEM((WIN, H), jnp.float32),
      pltpu.SemaphoreType.DMA,
      pltpu.SemaphoreType.DMA,
      pltpu.SemaphoreType.DMA,
      pltpu.SemaphoreType.DMA,
      pltpu.SemaphoreType.DMA,
      pltpu.SemaphoreType.DMA,
      pltpu.VMEM_SHARED((NP, H), jnp.float32),
  ]

  @functools.partial(
      pl.kernel,
      mesh=mesh,
      out_type=jax.ShapeDtypeStruct((2, NP, H), jnp.float32),
      scratch_types=list(_SCRATCH),
  )
  def _agg_sc(h_hbm, srcw_hbm, dstw_hbm, out_hbm, srcA, dstA, srcB, dstB,
              b0, b1, isemA, isemB, gs0, gs1, ss0, ss1, aggS):
    c = lax.axis_index("c")
    s = lax.axis_index("s")
    widx = c * 16 + s
    _agg_body(h_hbm, srcw_hbm, dstw_hbm, out_hbm, srcA, dstA, srcB, dstB,
              b0, b1, isemA, isemB, gs0, gs1, ss0, ss1, aggS, c, s, widx)

  @functools.partial(
      pl.kernel,
      mesh=mesh,
      out_type=(jax.ShapeDtypeStruct((2, NP, H), jnp.float32),
                jax.ShapeDtypeStruct((2, NP, H), jnp.float32)),
      scratch_types=list(_SCRATCH),
  )
  def _agg_deg_sc(h_hbm, srcw_hbm, dstw_hbm, agg_hbm, deg_hbm, srcA, dstA,
                  srcB, dstB, b0, b1, isemA, isemB, gs0, gs1, ss0, ss1, aggS):
    c = lax.axis_index("c")
    s = lax.axis_index("s")
    widx = c * 16 + s
    _agg_body(h_hbm, srcw_hbm, dstw_hbm, agg_hbm, srcA, dstA, srcB, dstB,
              b0, b1, isemA, isemB, gs0, gs1, ss0, ss1, aggS, c, s, widx)

    plsc.subcore_barrier()

    def _ob(i, carry):
        for k in range(H // 16):
            b1[i, pl.ds(k * 16, 16)] = jnp.ones((16,), jnp.float32)
        return carry

    lax.fori_loop(0, WIN, _ob, 0)

    def _zb(i, carry):
        for k in range(H // 16):
            b0[i, pl.ds(k * 16, 16)] = jnp.zeros((16,), jnp.float32)
        return carry

    lax.fori_loop(0, WIN, _zb, 0)

    hz = [pltpu.async_copy(b0, aggS.at[pl.ds(s * RPT + t * WIN, WIN)], ss0)
          for t in range(RPT // WIN)]
    for h in hz:
        h.wait()
    hi = pltpu.async_copy(dstw_hbm.at[widx, pl.ds(0, CH)], dstA, isemA)
    hi.wait()

    plsc.subcore_barrier()

    qsem = (gs0, gs1, ss0, ss1)

    def _dproc(dstc):
        hs = []
        for w in range(CH):
            if w >= 4:
                hs[w - 4].wait()
            hs.append(pltpu.async_copy(b1, aggS.at[dstc.at[w]], qsem[w % 4],
                                       add=True))
        for w in range(CH - 4, CH):
            hs[w].wait()

    def _dstep(t, carry):
        c0 = t * 2 * CH
        hb = pltpu.async_copy(dstw_hbm.at[widx, pl.ds(c0 + CH, CH)], dstB,
                              isemB)
        _dproc(dstA)
        hb.wait()
        nxt = jnp.minimum(c0 + 2 * CH, NWIN - CH)
        ha = pltpu.async_copy(dstw_hbm.at[widx, pl.ds(nxt, CH)], dstA, isemA)
        _dproc(dstB)
        ha.wait()
        return carry

    lax.fori_loop(0, NWIN // (2 * CH), _dstep, 0)

    plsc.subcore_barrier()
    pltpu.sync_copy(aggS.at[pl.ds(s * RPT, RPT)],
                    deg_hbm.at[c, pl.ds(s * RPT, RPT)])

  return _agg_sc, _agg_deg_sc


def _mm_body(x_ref, w_ref, b_ref, o_ref):
    o_ref[...] = (
        jnp.dot(x_ref[...], w_ref[...], preferred_element_type=jnp.float32)
        + b_ref[...]
    )


def _mm_call(x, w, b):
    return pl.pallas_call(
        _mm_body,
        grid=(NBLK,),
        in_specs=[
            pl.BlockSpec((BLK, D), lambda j: (j, 0)),
            pl.BlockSpec((D, H), lambda j: (0, 0)),
            pl.BlockSpec((1, H), lambda j: (0, 0)),
        ],
        out_specs=pl.BlockSpec((BLK, H), lambda j: (j, 0)),
        out_shape=jax.ShapeDtypeStruct((N, H), jnp.float32),
    )(x, w, b)


def _mid0_body(a0, a1, d0, d1, g_ref, be_ref, w_ref, b_ref, o_ref, oi_ref):
    a = a0[0] + a1[0]
    d = d0[0] + d1[0]
    inv = 1.0 / jnp.maximum(d, 1.0)
    oi_ref[...] = inv
    f = jnp.maximum(a * inv * (g_ref[...] * ISQ) + be_ref[...], 0.0)
    o_ref[...] = (
        jnp.dot(f, w_ref[...], preferred_element_type=jnp.float32) + b_ref[...]
    )


def _mid0_call(parts, degp, g, be, w, b):
    return pl.pallas_call(
        _mid0_body,
        grid=(NBLK,),
        in_specs=[
            pl.BlockSpec((1, BLK, H), lambda j: (0, j, 0)),
            pl.BlockSpec((1, BLK, H), lambda j: (1, j, 0)),
            pl.BlockSpec((1, BLK, H), lambda j: (0, j, 0)),
            pl.BlockSpec((1, BLK, H), lambda j: (1, j, 0)),
            pl.BlockSpec((1, H), lambda j: (0, 0)),
            pl.BlockSpec((1, H), lambda j: (0, 0)),
            pl.BlockSpec((H, H), lambda j: (0, 0)),
            pl.BlockSpec((1, H), lambda j: (0, 0)),
        ],
        out_specs=[
            pl.BlockSpec((BLK, H), lambda j: (j, 0)),
            pl.BlockSpec((BLK, H), lambda j: (j, 0)),
        ],
        out_shape=[
            jax.ShapeDtypeStruct((N, H), jnp.float32),
            jax.ShapeDtypeStruct((N, H), jnp.float32),
        ],
    )(parts, parts, degp, degp, g.reshape(1, H), be.reshape(1, H), w,
      b.reshape(1, H))


def _mid_body(a0, a1, iv, g_ref, be_ref, w_ref, b_ref, o_ref):
    a = a0[0] + a1[0]
    f = jnp.maximum(a * iv[...] * (g_ref[...] * ISQ) + be_ref[...], 0.0)
    o_ref[...] = (
        jnp.dot(f, w_ref[...], preferred_element_type=jnp.float32) + b_ref[...]
    )


def _mid_call(parts, invb, g, be, w, b):
    return pl.pallas_call(
        _mid_body,
        grid=(NBLK,),
        in_specs=[
            pl.BlockSpec((1, BLK, H), lambda j: (0, j, 0)),
            pl.BlockSpec((1, BLK, H), lambda j: (1, j, 0)),
            pl.BlockSpec((BLK, H), lambda j: (j, 0)),
            pl.BlockSpec((1, H), lambda j: (0, 0)),
            pl.BlockSpec((1, H), lambda j: (0, 0)),
            pl.BlockSpec((H, H), lambda j: (0, 0)),
            pl.BlockSpec((1, H), lambda j: (0, 0)),
        ],
        out_specs=pl.BlockSpec((BLK, H), lambda j: (j, 0)),
        out_shape=jax.ShapeDtypeStruct((N, H), jnp.float32),
    )(parts, parts, invb, g.reshape(1, H), be.reshape(1, H), w,
      b.reshape(1, H))


def _head_body(a0, a1, iv, g_ref, be_ref, bt, wh1, bh1, wh2, bh2, o_ref,
               ps, cs):
    j = pl.program_id(0)

    @pl.when(j == 0)
    def _():
        ps[...] = jnp.zeros_like(ps)
        cs[...] = jnp.zeros_like(cs)

    a = a0[0] + a1[0]
    f = jnp.maximum(a * iv[...] * (g_ref[...] * ISQ) + be_ref[...], 0.0)
    bids = bt[0]
    mask = jnp.equal(
        lax.broadcasted_iota(jnp.int32, (G, BLK), 0),
        jnp.broadcast_to(bids, (G, BLK)),
    ).astype(jnp.float32)
    ps[...] += jnp.dot(mask, f, preferred_element_type=jnp.float32)
    cs[...] += jnp.dot(mask, jnp.ones((BLK, H), jnp.float32),
                       preferred_element_type=jnp.float32)

    @pl.when(j == NBLK - 1)
    def _():
        pooled = ps[...] / jnp.maximum(cs[...], 1.0)
        hdn = jnp.maximum(
            jnp.dot(pooled, wh1[...], preferred_element_type=jnp.float32)
            + bh1[...], 0.0)
        o_ref[...] = (
            jnp.dot(hdn, wh2[...], preferred_element_type=jnp.float32)
            + bh2[...]
        )


def _head_call(parts, invb, g, be, batchr, wh1, bh1, wh2p, bh2p):
    return pl.pallas_call(
        _head_body,
        grid=(NBLK,),
        in_specs=[
            pl.BlockSpec((1, BLK, H), lambda j: (0, j, 0)),
            pl.BlockSpec((1, BLK, H), lambda j: (1, j, 0)),
            pl.BlockSpec((BLK, H), lambda j: (j, 0)),
            pl.BlockSpec((1, H), lambda j: (0, 0)),
            pl.BlockSpec((1, H), lambda j: (0, 0)),
            pl.BlockSpec((1, 1, BLK), lambda j: (j, 0, 0)),
            pl.BlockSpec((H, H), lambda j: (0, 0)),
            pl.BlockSpec((1, H), lambda j: (0, 0)),
            pl.BlockSpec((H, H), lambda j: (0, 0)),
            pl.BlockSpec((1, H), lambda j: (0, 0)),
        ],
        out_specs=pl.BlockSpec((G, H), lambda j: (0, 0)),
        out_shape=jax.ShapeDtypeStruct((G, H), jnp.float32),
        scratch_shapes=[
            pltpu.VMEM((G, H), jnp.float32),
            pltpu.VMEM((G, H), jnp.float32),
        ],
    )(parts, parts, invb, g.reshape(1, H), be.reshape(1, H), batchr,
      wh1, bh1.reshape(1, H), wh2p, bh2p.reshape(1, H))


def kernel(x, edge_index, batch, W0, b0, g0, be0, W1, b1, g1, be1, W2, b2,
           g2, be2, W3, b3, g3, be3, Wh1, bh1, Wh2, bh2):
    src = edge_index[0]
    dst = edge_index[1]
    npad = EPAD - E
    pidx = jnp.arange(npad, dtype=jnp.int32)
    srcp = jnp.concatenate([src, pidx % N]).reshape(NW, NWIN, WIN)
    dstp = jnp.concatenate([dst, N + pidx % (NP - N)]).reshape(NW, NWIN, WIN)

    agg_sc, agg_deg_sc = _sc_kernels()
    h = _mm_call(x, W0, b0.reshape(1, H))

    layer = [(g0, be0), (g1, be1), (g2, be2), (g3, be3)]
    nxt = [(W1, b1), (W2, b2), (W3, b3)]
    for i in range(3):
        g, be = layer[i]
        w, b = nxt[i]
        if i == 0:
            parts, degp = agg_deg_sc(h, srcp, dstp)
            h, invb = _mid0_call(parts, degp, g, be, w, b)
        else:
            parts = agg_sc(h, srcp, dstp)
            h = _mid_call(parts, invb, g, be, w, b)

    parts = agg_sc(h, srcp, dstp)
    batchr = batch.reshape(NBLK, 1, BLK)
    wh2p = jnp.pad(Wh2, ((0, 0), (0, H - C)))
    bh2p = jnp.pad(bh2, (0, H - C))
    out = _head_call(parts, invb, layer[3][0], layer[3][1], batchr, Wh1,
                     bh1, wh2p, bh2p)
    return out[:, :C]
